# baseline (speedup 1.0000x reference)
import jax
import jax.numpy as jnp
from jax import lax
from jax.experimental import pallas as pl
from jax.experimental.pallas import tpu as pltpu
from jax.experimental.pallas import tpu_sc as plsc

_N = 10000
_E = 320000
_D = 128
_HOR = 24

_NC = 2
_NS = 16
_NW = _NC * _NS
_L = 16

_K = 128
_C = 80
_EPT = _K * _C
_EPAD = _EPT * _NW
_RPT = _N // _NS
_NBUF = 2


def _spmm_body(x_hbm, src_hbm, dst_hbm, w_hbm, out_hbm,
               src_t, bufs, dstc, wc, gsems, ssems, agg_sh):
    c = lax.axis_index("c")
    s = lax.axis_index("s")
    wid = s * _NC + c

    pltpu.sync_copy(src_hbm.at[wid], src_t)

    def _zrow(i, carry):
        for d in range(_D // _L):
            bufs[0][i, pl.ds(d * _L, _L)] = jnp.zeros((_L,), jnp.float32)
        return carry

    lax.fori_loop(0, _K, _zrow, 0)
    base = s * _RPT
    for j in range(_RPT // _K):
        pltpu.sync_copy(bufs[0].at[pl.ds(0, _K)],
                        agg_sh.at[pl.ds(base + j * _K, _K)])
    rem = _RPT % _K
    if rem:
        pltpu.sync_copy(bufs[0].at[pl.ds(0, rem)],
                        agg_sh.at[pl.ds(base + (_RPT // _K) * _K, rem)])
    plsc.subcore_barrier()

    def _scale(buf, wbuf):
        @plsc.parallel_loop(0, _K // _L, 1)
        def _group(g):
            w16 = wbuf[pl.ds(g * _L, _L)]
            for j in range(_L):
                e = g * _L + j
                wv = lax.broadcast(w16[j], (_L,))
                for d in range(_D // _L):
                    sl = pl.ds(d * _L, _L)
                    buf[e, sl] = buf[e, sl] * wv

    def _fire_gather(ci, b):
        pltpu.async_copy(x_hbm.at[src_t.at[ci]], bufs[b], gsems[b])
        pltpu.async_copy(dst_hbm.at[wid, ci], dstc[b], gsems[b])
        pltpu.async_copy(w_hbm.at[wid, ci], wc[b], gsems[b])

    def _wait_gather(ci, b):
        pltpu.make_async_copy(x_hbm.at[src_t.at[ci]], bufs[b],
                              gsems[b]).wait()
        pltpu.make_async_copy(dst_hbm.at[wid, ci], dstc[b], gsems[b]).wait()
        pltpu.make_async_copy(w_hbm.at[wid, ci], wc[b], gsems[b]).wait()

    def _wait_scatter(b):
        pltpu.make_async_copy(bufs[b], agg_sh.at[dstc[b]], ssems[b]).wait()

    for b in range(_NBUF):
        _fire_gather(b, b)

    def _chunk(i, carry):
        for b in range(_NBUF):
            bprev = (b + _NBUF - 1) % _NBUF

            @pl.when(i % _NBUF == b)
            def _():
                _wait_gather(i, b)
                _scale(bufs[b], wc[b])
                pltpu.async_copy(bufs[b], agg_sh.at[dstc[b]], ssems[b],
                                 add=True)

                @pl.when((i >= 1) & (i + _NBUF - 1 < _C))
                def _():
                    _wait_scatter(bprev)
                    _fire_gather(i + _NBUF - 1, bprev)

        return carry

    lax.fori_loop(0, _C, _chunk, 0)

    for ci in range(_C - _NBUF, _C):
        _wait_scatter(ci % _NBUF)

    plsc.subcore_barrier()
    pltpu.sync_copy(agg_sh.at[pl.ds(base, _RPT)],
                    out_hbm.at[c, pl.ds(base, _RPT)])


_spmm = pl.kernel(
    _spmm_body,
    out_type=jax.ShapeDtypeStruct((_NC, _N, _D), jnp.float32),
    mesh=plsc.VectorSubcoreMesh(core_axis_name="c", subcore_axis_name="s"),
    compiler_params=pltpu.CompilerParams(use_tc_tiling_on_sc=False),
    scratch_types=[
        pltpu.VMEM((_C, _K), jnp.int32),
        tuple(pltpu.VMEM((_K, _D), jnp.float32) for _ in range(_NBUF)),
        tuple(pltpu.VMEM((_K,), jnp.int32) for _ in range(_NBUF)),
        tuple(pltpu.VMEM((_K,), jnp.float32) for _ in range(_NBUF)),
        tuple(pltpu.SemaphoreType.DMA for _ in range(_NBUF)),
        tuple(pltpu.SemaphoreType.DMA for _ in range(_NBUF)),
        pltpu.VMEM_SHARED((_N, _D), jnp.float32),
    ],
)


def _dense_body(p_ref, x_ref, wrel_ref, wroot_ref, b_ref, o_ref):
    agg = p_ref[0] + p_ref[1]
    h = (jnp.dot(agg, wrel_ref[...], preferred_element_type=jnp.float32)
         + jnp.dot(x_ref[...], wroot_ref[...], preferred_element_type=jnp.float32)
         + b_ref[...])
    o_ref[...] = jnp.maximum(h, 0.0)


def _head_body(p_ref, x_ref, wrel_ref, wroot_ref, b_ref, wfc_ref, bfc_ref,
               wlast_ref, blast_ref, o_ref):
    agg = p_ref[0] + p_ref[1]
    h = (jnp.dot(agg, wrel_ref[...], preferred_element_type=jnp.float32)
         + jnp.dot(x_ref[...], wroot_ref[...], preferred_element_type=jnp.float32)
         + b_ref[...])
    h = jnp.maximum(h, 0.0)
    h = jnp.maximum(
        jnp.dot(h, wfc_ref[...], preferred_element_type=jnp.float32) + bfc_ref[...],
        0.0)
    o_ref[...] = (jnp.dot(h, wlast_ref[...], preferred_element_type=jnp.float32)
                  + blast_ref[...])


_RB = 1000


def _dense(parts, x, w_rel, w_root, b):
    grid = (_N // _RB,)
    return pl.pallas_call(
        _dense_body,
        grid=grid,
        in_specs=[
            pl.BlockSpec((_NC, _RB, _D), lambda i: (0, i, 0)),
            pl.BlockSpec((_RB, _D), lambda i: (i, 0)),
            pl.BlockSpec((_D, _D), lambda i: (0, 0)),
            pl.BlockSpec((_D, _D), lambda i: (0, 0)),
            pl.BlockSpec((1, _D), lambda i: (0, 0)),
        ],
        out_specs=pl.BlockSpec((_RB, _D), lambda i: (i, 0)),
        out_shape=jax.ShapeDtypeStruct((_N, _D), jnp.float32),
    )(parts, x, w_rel, w_root, b.reshape(1, _D))


def _head(parts, x, w_rel, w_root, b, w_fc, b_fc, w_last_p, b_last_p):
    grid = (_N // _RB,)
    return pl.pallas_call(
        _head_body,
        grid=grid,
        in_specs=[
            pl.BlockSpec((_NC, _RB, _D), lambda i: (0, i, 0)),
            pl.BlockSpec((_RB, _D), lambda i: (i, 0)),
            pl.BlockSpec((_D, _D), lambda i: (0, 0)),
            pl.BlockSpec((_D, _D), lambda i: (0, 0)),
            pl.BlockSpec((1, _D), lambda i: (0, 0)),
            pl.BlockSpec((_D, _D), lambda i: (0, 0)),
            pl.BlockSpec((1, _D), lambda i: (0, 0)),
            pl.BlockSpec((_D, _D), lambda i: (0, 0)),
            pl.BlockSpec((1, _D), lambda i: (0, 0)),
        ],
        out_specs=pl.BlockSpec((_RB, _D), lambda i: (i, 0)),
        out_shape=jax.ShapeDtypeStruct((_N, _D), jnp.float32),
    )(parts, x, w_rel, w_root, b.reshape(1, _D), w_fc, b_fc.reshape(1, _D),
      w_last_p, b_last_p)


def kernel(x, edge_index, edge_weights, W_rel1, b_rel1, W_root1,
           W_rel2, b_rel2, W_root2, W_fc, b_fc, W_last, b_last):
    pad = _EPAD - _E
    src = jnp.concatenate(
        [edge_index[0].astype(jnp.int32), jnp.zeros((pad,), jnp.int32)]
    ).reshape(_NW, _C, _K)
    dst = jnp.concatenate(
        [edge_index[1].astype(jnp.int32), jnp.zeros((pad,), jnp.int32)]
    ).reshape(_NW, _C, _K)
    w = jnp.concatenate(
        [edge_weights, jnp.zeros((pad,), jnp.float32)]
    ).reshape(_NW, _C, _K)

    w_last_p = jnp.zeros((_D, _D), jnp.float32).at[:, :_HOR].set(W_last)
    b_last_p = jnp.zeros((1, _D), jnp.float32).at[0, :_HOR].set(b_last)

    p1 = _spmm(x, src, dst, w)
    h1 = _dense(p1, x, W_rel1, W_root1, b_rel1)
    p2 = _spmm(h1, src, dst, w)
    h2 = _dense(p2, h1, W_rel2, W_root2, b_rel2)
    p3 = _spmm(h2, src, dst, w)
    out = _head(p3, h2, W_rel2, W_root2, b_rel2, W_fc, b_fc,
                w_last_p, b_last_p)
    return out[:, :_HOR]


# speedup vs baseline: 1.9323x; 1.9323x over previous
import jax
import jax.numpy as jnp
from jax import lax
from jax.experimental import pallas as pl
from jax.experimental.pallas import tpu as pltpu
from jax.experimental.pallas import tpu_sc as plsc

_N = 10000
_E = 320000
_D = 128
_HOR = 24

_NC = 2
_NS = 16
_NW = _NC * _NS
_L = 16

_K = 96
_C = 105
_EPT = _K * _C
_EPAD = _EPT * _NW
_RPT = _N // _NS
_NBUF = 3


def _spmm_body(x_hbm, src_hbm, dst_hbm, w_hbm, out_hbm,
               src_t, bufs, dstc, wc, gsems, ssems, agg_sh):
    c = lax.axis_index("c")
    s = lax.axis_index("s")
    wid = s * _NC + c

    pltpu.sync_copy(src_hbm.at[wid], src_t)

    def _zrow(i, carry):
        for d in range(_D // _L):
            bufs[0][i, pl.ds(d * _L, _L)] = jnp.zeros((_L,), jnp.float32)
        return carry

    lax.fori_loop(0, _K, _zrow, 0)
    base = s * _RPT
    for j in range(_RPT // _K):
        pltpu.sync_copy(bufs[0].at[pl.ds(0, _K)],
                        agg_sh.at[pl.ds(base + j * _K, _K)])
    rem = _RPT % _K
    if rem:
        pltpu.sync_copy(bufs[0].at[pl.ds(0, rem)],
                        agg_sh.at[pl.ds(base + (_RPT // _K) * _K, rem)])
    plsc.subcore_barrier()

    def _scale(buf, wbuf):
        @plsc.parallel_loop(0, _K // _L, 1)
        def _group(g):
            w16 = wbuf[pl.ds(g * _L, _L)]
            for j in range(_L):
                e = g * _L + j
                wv = lax.broadcast(w16[j], (_L,))
                for d in range(_D // _L):
                    sl = pl.ds(d * _L, _L)
                    buf[e, sl] = buf[e, sl] * wv

    def _fire_gather(ci, b):
        pltpu.async_copy(x_hbm.at[src_t.at[ci]], bufs[b], gsems[b])
        pltpu.async_copy(dst_hbm.at[wid, ci], dstc[b], gsems[b])
        pltpu.async_copy(w_hbm.at[wid, ci], wc[b], gsems[b])

    def _wait_gather(ci, b):
        pltpu.make_async_copy(x_hbm.at[src_t.at[ci]], bufs[b],
                              gsems[b]).wait()
        pltpu.make_async_copy(dst_hbm.at[wid, ci], dstc[b], gsems[b]).wait()
        pltpu.make_async_copy(w_hbm.at[wid, ci], wc[b], gsems[b]).wait()

    def _wait_scatter(b):
        pltpu.make_async_copy(bufs[b], agg_sh.at[dstc[b]], ssems[b]).wait()

    for b in range(_NBUF):
        _fire_gather(b, b)

    def _chunk(i, carry):
        for b in range(_NBUF):
            bprev = (b + _NBUF - 1) % _NBUF

            @pl.when(i % _NBUF == b)
            def _():
                _wait_gather(i, b)
                _scale(bufs[b], wc[b])
                pltpu.async_copy(bufs[b], agg_sh.at[dstc[b]], ssems[b],
                                 add=True)

                @pl.when((i >= 1) & (i + _NBUF - 1 < _C))
                def _():
                    _wait_scatter(bprev)
                    _fire_gather(i + _NBUF - 1, bprev)

        return carry

    lax.fori_loop(0, _C, _chunk, 0)

    for ci in range(_C - _NBUF, _C):
        _wait_scatter(ci % _NBUF)

    plsc.subcore_barrier()
    pltpu.sync_copy(agg_sh.at[pl.ds(base, _RPT)],
                    out_hbm.at[c, pl.ds(base, _RPT)])


_spmm = pl.kernel(
    _spmm_body,
    out_type=jax.ShapeDtypeStruct((_NC, _N, _D), jnp.float32),
    mesh=plsc.VectorSubcoreMesh(core_axis_name="c", subcore_axis_name="s"),
    compiler_params=pltpu.CompilerParams(use_tc_tiling_on_sc=False),
    scratch_types=[
        pltpu.VMEM((_C, _K), jnp.int32),
        tuple(pltpu.VMEM((_K, _D), jnp.float32) for _ in range(_NBUF)),
        tuple(pltpu.VMEM((_K,), jnp.int32) for _ in range(_NBUF)),
        tuple(pltpu.VMEM((_K,), jnp.float32) for _ in range(_NBUF)),
        tuple(pltpu.SemaphoreType.DMA for _ in range(_NBUF)),
        tuple(pltpu.SemaphoreType.DMA for _ in range(_NBUF)),
        pltpu.VMEM_SHARED((_N, _D), jnp.float32),
    ],
)


def _dense_body(p_ref, x_ref, wrel_ref, wroot_ref, b_ref, o_ref):
    agg = p_ref[0] + p_ref[1]
    h = (jnp.dot(agg, wrel_ref[...], preferred_element_type=jnp.float32)
         + jnp.dot(x_ref[...], wroot_ref[...], preferred_element_type=jnp.float32)
         + b_ref[...])
    o_ref[...] = jnp.maximum(h, 0.0)


def _head_body(p_ref, x_ref, wrel_ref, wroot_ref, b_ref, wfc_ref, bfc_ref,
               wlast_ref, blast_ref, o_ref):
    agg = p_ref[0] + p_ref[1]
    h = (jnp.dot(agg, wrel_ref[...], preferred_element_type=jnp.float32)
         + jnp.dot(x_ref[...], wroot_ref[...], preferred_element_type=jnp.float32)
         + b_ref[...])
    h = jnp.maximum(h, 0.0)
    h = jnp.maximum(
        jnp.dot(h, wfc_ref[...], preferred_element_type=jnp.float32) + bfc_ref[...],
        0.0)
    o_ref[...] = (jnp.dot(h, wlast_ref[...], preferred_element_type=jnp.float32)
                  + blast_ref[...])


_RB = 1000


def _dense(parts, x, w_rel, w_root, b):
    grid = (_N // _RB,)
    return pl.pallas_call(
        _dense_body,
        grid=grid,
        in_specs=[
            pl.BlockSpec((_NC, _RB, _D), lambda i: (0, i, 0)),
            pl.BlockSpec((_RB, _D), lambda i: (i, 0)),
            pl.BlockSpec((_D, _D), lambda i: (0, 0)),
            pl.BlockSpec((_D, _D), lambda i: (0, 0)),
            pl.BlockSpec((1, _D), lambda i: (0, 0)),
        ],
        out_specs=pl.BlockSpec((_RB, _D), lambda i: (i, 0)),
        out_shape=jax.ShapeDtypeStruct((_N, _D), jnp.float32),
    )(parts, x, w_rel, w_root, b.reshape(1, _D))


def _head(parts, x, w_rel, w_root, b, w_fc, b_fc, w_last_p, b_last_p):
    grid = (_N // _RB,)
    return pl.pallas_call(
        _head_body,
        grid=grid,
        in_specs=[
            pl.BlockSpec((_NC, _RB, _D), lambda i: (0, i, 0)),
            pl.BlockSpec((_RB, _D), lambda i: (i, 0)),
            pl.BlockSpec((_D, _D), lambda i: (0, 0)),
            pl.BlockSpec((_D, _D), lambda i: (0, 0)),
            pl.BlockSpec((1, _D), lambda i: (0, 0)),
            pl.BlockSpec((_D, _D), lambda i: (0, 0)),
            pl.BlockSpec((1, _D), lambda i: (0, 0)),
            pl.BlockSpec((_D, _D), lambda i: (0, 0)),
            pl.BlockSpec((1, _D), lambda i: (0, 0)),
        ],
        out_specs=pl.BlockSpec((_RB, _D), lambda i: (i, 0)),
        out_shape=jax.ShapeDtypeStruct((_N, _D), jnp.float32),
    )(parts, x, w_rel, w_root, b.reshape(1, _D), w_fc, b_fc.reshape(1, _D),
      w_last_p, b_last_p)


def kernel(x, edge_index, edge_weights, W_rel1, b_rel1, W_root1,
           W_rel2, b_rel2, W_root2, W_fc, b_fc, W_last, b_last):
    pad = _EPAD - _E
    src = jnp.concatenate(
        [edge_index[0].astype(jnp.int32), jnp.zeros((pad,), jnp.int32)]
    ).reshape(_NW, _C, _K)
    dst = jnp.concatenate(
        [edge_index[1].astype(jnp.int32), jnp.zeros((pad,), jnp.int32)]
    ).reshape(_NW, _C, _K)
    w = jnp.concatenate(
        [edge_weights, jnp.zeros((pad,), jnp.float32)]
    ).reshape(_NW, _C, _K)

    w_last_p = jnp.zeros((_D, _D), jnp.float32).at[:, :_HOR].set(W_last)
    b_last_p = jnp.zeros((1, _D), jnp.float32).at[0, :_HOR].set(b_last)

    p1 = _spmm(x, src, dst, w)
    h1 = _dense(p1, x, W_rel1, W_root1, b_rel1)
    p2 = _spmm(h1, src, dst, w)
    h2 = _dense(p2, h1, W_rel2, W_root2, b_rel2)
    p3 = _spmm(h2, src, dst, w)
    out = _head(p3, h2, W_rel2, W_root2, b_rel2, W_fc, b_fc,
                w_last_p, b_last_p)
    return out[:, :_HOR]


# speedup vs baseline: 1.9563x; 1.0125x over previous
import jax
import jax.numpy as jnp
from jax import lax
from jax.experimental import pallas as pl
from jax.experimental.pallas import tpu as pltpu
from jax.experimental.pallas import tpu_sc as plsc

_N = 10000
_E = 320000
_D = 128
_HOR = 24

_NC = 2
_NS = 16
_NW = _NC * _NS
_L = 16

_K = 96
_C = 105
_EPT = _K * _C
_EPAD = _EPT * _NW
_RPT = _N // _NS
_NBUF = 3


def _spmm_body(x_hbm, src_hbm, dst_hbm, w_hbm, out_hbm,
               src_t, bufs, dstc, wc, gsems, ssems, agg_sh):
    c = lax.axis_index("c")
    s = lax.axis_index("s")
    wid = s * _NC + c

    pltpu.sync_copy(src_hbm.at[wid], src_t)

    def _zrow(i, carry):
        for d in range(_D // _L):
            bufs[0][i, pl.ds(d * _L, _L)] = jnp.zeros((_L,), jnp.float32)
        return carry

    lax.fori_loop(0, _K, _zrow, 0)
    base = s * _RPT
    for j in range(_RPT // _K):
        pltpu.sync_copy(bufs[0].at[pl.ds(0, _K)],
                        agg_sh.at[pl.ds(base + j * _K, _K)])
    rem = _RPT % _K
    if rem:
        pltpu.sync_copy(bufs[0].at[pl.ds(0, rem)],
                        agg_sh.at[pl.ds(base + (_RPT // _K) * _K, rem)])
    plsc.subcore_barrier()

    def _scale(buf, wbuf):
        def _group(g, carry2):
            w16 = wbuf[pl.ds(g * _L, _L)]
            for j in range(_L):
                e = g * _L + j
                wv = lax.broadcast(w16[j], (_L,))
                for d in range(_D // _L):
                    sl = pl.ds(d * _L, _L)
                    buf[e, sl] = buf[e, sl] * wv
            return carry2

        lax.fori_loop(0, _K // _L, _group, 0)

    def _fire_gather(ci, b):
        pltpu.async_copy(x_hbm.at[src_t.at[ci]], bufs[b], gsems[b])
        pltpu.async_copy(dst_hbm.at[wid, ci], dstc[b], gsems[b])
        pltpu.async_copy(w_hbm.at[wid, ci], wc[b], gsems[b])

    def _wait_gather(ci, b):
        pltpu.make_async_copy(x_hbm.at[src_t.at[ci]], bufs[b],
                              gsems[b]).wait()
        pltpu.make_async_copy(dst_hbm.at[wid, ci], dstc[b], gsems[b]).wait()
        pltpu.make_async_copy(w_hbm.at[wid, ci], wc[b], gsems[b]).wait()

    def _wait_scatter(b):
        pltpu.make_async_copy(bufs[b], agg_sh.at[dstc[b]], ssems[b]).wait()

    for b in range(_NBUF):
        _fire_gather(b, b)

    def _chunk(i, carry):
        for b in range(_NBUF):
            bprev = (b + _NBUF - 1) % _NBUF

            @pl.when(i % _NBUF == b)
            def _():
                _wait_gather(i, b)
                _scale(bufs[b], wc[b])
                pltpu.async_copy(bufs[b], agg_sh.at[dstc[b]], ssems[b],
                                 add=True)

                @pl.when((i >= 1) & (i + _NBUF - 1 < _C))
                def _():
                    _wait_scatter(bprev)
                    _fire_gather(i + _NBUF - 1, bprev)

        return carry

    lax.fori_loop(0, _C, _chunk, 0)

    for ci in range(_C - _NBUF, _C):
        _wait_scatter(ci % _NBUF)

    plsc.subcore_barrier()
    pltpu.sync_copy(agg_sh.at[pl.ds(base, _RPT)],
                    out_hbm.at[c, pl.ds(base, _RPT)])


_spmm = pl.kernel(
    _spmm_body,
    out_type=jax.ShapeDtypeStruct((_NC, _N, _D), jnp.float32),
    mesh=plsc.VectorSubcoreMesh(core_axis_name="c", subcore_axis_name="s"),
    compiler_params=pltpu.CompilerParams(use_tc_tiling_on_sc=False),
    scratch_types=[
        pltpu.VMEM((_C, _K), jnp.int32),
        tuple(pltpu.VMEM((_K, _D), jnp.float32) for _ in range(_NBUF)),
        tuple(pltpu.VMEM((_K,), jnp.int32) for _ in range(_NBUF)),
        tuple(pltpu.VMEM((_K,), jnp.float32) for _ in range(_NBUF)),
        tuple(pltpu.SemaphoreType.DMA for _ in range(_NBUF)),
        tuple(pltpu.SemaphoreType.DMA for _ in range(_NBUF)),
        pltpu.VMEM_SHARED((_N, _D), jnp.float32),
    ],
)


def _dense_body(p_ref, x_ref, wrel_ref, wroot_ref, b_ref, o_ref):
    agg = p_ref[0] + p_ref[1]
    h = (jnp.dot(agg, wrel_ref[...], preferred_element_type=jnp.float32)
         + jnp.dot(x_ref[...], wroot_ref[...], preferred_element_type=jnp.float32)
         + b_ref[...])
    o_ref[...] = jnp.maximum(h, 0.0)


def _head_body(p_ref, x_ref, wrel_ref, wroot_ref, b_ref, wfc_ref, bfc_ref,
               wlast_ref, blast_ref, o_ref):
    agg = p_ref[0] + p_ref[1]
    h = (jnp.dot(agg, wrel_ref[...], preferred_element_type=jnp.float32)
         + jnp.dot(x_ref[...], wroot_ref[...], preferred_element_type=jnp.float32)
         + b_ref[...])
    h = jnp.maximum(h, 0.0)
    h = jnp.maximum(
        jnp.dot(h, wfc_ref[...], preferred_element_type=jnp.float32) + bfc_ref[...],
        0.0)
    o_ref[...] = (jnp.dot(h, wlast_ref[...], preferred_element_type=jnp.float32)
                  + blast_ref[...])


_RB = 1000


def _dense(parts, x, w_rel, w_root, b):
    grid = (_N // _RB,)
    return pl.pallas_call(
        _dense_body,
        grid=grid,
        in_specs=[
            pl.BlockSpec((_NC, _RB, _D), lambda i: (0, i, 0)),
            pl.BlockSpec((_RB, _D), lambda i: (i, 0)),
            pl.BlockSpec((_D, _D), lambda i: (0, 0)),
            pl.BlockSpec((_D, _D), lambda i: (0, 0)),
            pl.BlockSpec((1, _D), lambda i: (0, 0)),
        ],
        out_specs=pl.BlockSpec((_RB, _D), lambda i: (i, 0)),
        out_shape=jax.ShapeDtypeStruct((_N, _D), jnp.float32),
    )(parts, x, w_rel, w_root, b.reshape(1, _D))


def _head(parts, x, w_rel, w_root, b, w_fc, b_fc, w_last_p, b_last_p):
    grid = (_N // _RB,)
    return pl.pallas_call(
        _head_body,
        grid=grid,
        in_specs=[
            pl.BlockSpec((_NC, _RB, _D), lambda i: (0, i, 0)),
            pl.BlockSpec((_RB, _D), lambda i: (i, 0)),
            pl.BlockSpec((_D, _D), lambda i: (0, 0)),
            pl.BlockSpec((_D, _D), lambda i: (0, 0)),
            pl.BlockSpec((1, _D), lambda i: (0, 0)),
            pl.BlockSpec((_D, _D), lambda i: (0, 0)),
            pl.BlockSpec((1, _D), lambda i: (0, 0)),
            pl.BlockSpec((_D, _D), lambda i: (0, 0)),
            pl.BlockSpec((1, _D), lambda i: (0, 0)),
        ],
        out_specs=pl.BlockSpec((_RB, _D), lambda i: (i, 0)),
        out_shape=jax.ShapeDtypeStruct((_N, _D), jnp.float32),
    )(parts, x, w_rel, w_root, b.reshape(1, _D), w_fc, b_fc.reshape(1, _D),
      w_last_p, b_last_p)


def kernel(x, edge_index, edge_weights, W_rel1, b_rel1, W_root1,
           W_rel2, b_rel2, W_root2, W_fc, b_fc, W_last, b_last):
    pad = _EPAD - _E
    src = jnp.concatenate(
        [edge_index[0].astype(jnp.int32), jnp.zeros((pad,), jnp.int32)]
    ).reshape(_NW, _C, _K)
    dst = jnp.concatenate(
        [edge_index[1].astype(jnp.int32), jnp.zeros((pad,), jnp.int32)]
    ).reshape(_NW, _C, _K)
    w = jnp.concatenate(
        [edge_weights, jnp.zeros((pad,), jnp.float32)]
    ).reshape(_NW, _C, _K)

    w_last_p = jnp.zeros((_D, _D), jnp.float32).at[:, :_HOR].set(W_last)
    b_last_p = jnp.zeros((1, _D), jnp.float32).at[0, :_HOR].set(b_last)

    p1 = _spmm(x, src, dst, w)
    h1 = _dense(p1, x, W_rel1, W_root1, b_rel1)
    p2 = _spmm(h1, src, dst, w)
    h2 = _dense(p2, h1, W_rel2, W_root2, b_rel2)
    p3 = _spmm(h2, src, dst, w)
    out = _head(p3, h2, W_rel2, W_root2, b_rel2, W_fc, b_fc,
                w_last_p, b_last_p)
    return out[:, :_HOR]


# speedup vs baseline: 3.6762x; 1.8791x over previous
import jax
import jax.numpy as jnp
from jax import lax
from jax.experimental import pallas as pl
from jax.experimental.pallas import tpu as pltpu
from jax.experimental.pallas import tpu_sc as plsc

_N = 10000
_E = 320000
_D = 128
_HOR = 24

_NC = 2
_NS = 16
_NW = _NC * _NS
_L = 16

_K = 96
_C = 105
_EPT = _K * _C
_EPAD = _EPT * _NW
_RPT = _N // _NS
_NBUF = 3


def _spmm_body(x_hbm, src_hbm, dst_hbm, w_hbm, out_hbm,
               src_t, bufs, dstc, wc, gsems, ssems, agg_sh):
    c = lax.axis_index("c")
    s = lax.axis_index("s")
    wid = s * _NC + c

    pltpu.sync_copy(src_hbm.at[wid], src_t)

    def _zrow(i, carry):
        for d in range(_D // _L):
            bufs[0][i, pl.ds(d * _L, _L)] = jnp.zeros((_L,), jnp.float32)
        return carry

    lax.fori_loop(0, _K, _zrow, 0)
    base = s * _RPT
    for j in range(_RPT // _K):
        pltpu.sync_copy(bufs[0].at[pl.ds(0, _K)],
                        agg_sh.at[pl.ds(base + j * _K, _K)])
    rem = _RPT % _K
    if rem:
        pltpu.sync_copy(bufs[0].at[pl.ds(0, rem)],
                        agg_sh.at[pl.ds(base + (_RPT // _K) * _K, rem)])
    plsc.subcore_barrier()

    def _scale(buf, wbuf):
        def _group(g, carry2):
            w16 = wbuf[pl.ds(g * _L, _L)]
            for j in range(_L):
                e = g * _L + j
                wv = lax.broadcast(w16[j], (_L,))
                for d in range(_D // _L):
                    sl = pl.ds(d * _L, _L)
                    buf[e, sl] = buf[e, sl] * wv
            return carry2

        lax.fori_loop(0, _K // _L, _group, 0)

    def _fire_gather(ci, b):
        pltpu.async_copy(x_hbm.at[src_t.at[ci]], bufs[b], gsems[b])
        pltpu.async_copy(dst_hbm.at[wid, ci], dstc[b], gsems[b])
        pltpu.async_copy(w_hbm.at[wid, ci], wc[b], gsems[b])

    def _wait_gather(ci, b):
        pltpu.make_async_copy(x_hbm.at[src_t.at[ci]], bufs[b],
                              gsems[b]).wait()
        pltpu.make_async_copy(dst_hbm.at[wid, ci], dstc[b], gsems[b]).wait()
        pltpu.make_async_copy(w_hbm.at[wid, ci], wc[b], gsems[b]).wait()

    def _wait_scatter(b):
        pltpu.make_async_copy(bufs[b], agg_sh.at[dstc[b]], ssems[b]).wait()

    for b in range(_NBUF):
        _fire_gather(b, b)

    def _chunk(i, carry):
        for b in range(_NBUF):
            bprev = (b + _NBUF - 1) % _NBUF

            @pl.when(i % _NBUF == b)
            def _():
                _wait_gather(i, b)
                _scale(bufs[b], wc[b])
                pltpu.async_copy(bufs[b], agg_sh.at[dstc[b]], ssems[b],
                                 add=True)

                @pl.when((i >= 1) & (i + _NBUF - 1 < _C))
                def _():
                    _wait_scatter(bprev)
                    _fire_gather(i + _NBUF - 1, bprev)

        return carry

    lax.fori_loop(0, _C, _chunk, 0)

    for ci in range(_C - _NBUF, _C):
        _wait_scatter(ci % _NBUF)

    plsc.subcore_barrier()
    pltpu.sync_copy(agg_sh.at[pl.ds(base, _RPT)],
                    out_hbm.at[c, pl.ds(base, _RPT)])


_spmm = pl.kernel(
    _spmm_body,
    out_type=jax.ShapeDtypeStruct((_NC, _N, _D), jnp.float32),
    mesh=plsc.VectorSubcoreMesh(core_axis_name="c", subcore_axis_name="s"),
    compiler_params=pltpu.CompilerParams(use_tc_tiling_on_sc=False),
    scratch_types=[
        pltpu.VMEM((_C, _K), jnp.int32),
        tuple(pltpu.VMEM((_K, _D), jnp.float32) for _ in range(_NBUF)),
        tuple(pltpu.VMEM((_K,), jnp.int32) for _ in range(_NBUF)),
        tuple(pltpu.VMEM((_K,), jnp.float32) for _ in range(_NBUF)),
        tuple(pltpu.SemaphoreType.DMA for _ in range(_NBUF)),
        tuple(pltpu.SemaphoreType.DMA for _ in range(_NBUF)),
        pltpu.VMEM_SHARED((_N, _D), jnp.float32),
    ],
)


def _dense_body(p_ref, x_ref, wrel_ref, wroot_ref, b_ref, o_ref):
    agg = p_ref[0] + p_ref[1]
    h = (jnp.dot(agg, wrel_ref[...], preferred_element_type=jnp.float32)
         + jnp.dot(x_ref[...], wroot_ref[...], preferred_element_type=jnp.float32)
         + b_ref[...])
    o_ref[...] = jnp.maximum(h, 0.0)


def _head_body(p_ref, x_ref, wrel_ref, wroot_ref, b_ref, wfc_ref, bfc_ref,
               wlast_ref, blast_ref, o_ref):
    agg = p_ref[0] + p_ref[1]
    h = (jnp.dot(agg, wrel_ref[...], preferred_element_type=jnp.float32)
         + jnp.dot(x_ref[...], wroot_ref[...], preferred_element_type=jnp.float32)
         + b_ref[...])
    h = jnp.maximum(h, 0.0)
    h = jnp.maximum(
        jnp.dot(h, wfc_ref[...], preferred_element_type=jnp.float32) + bfc_ref[...],
        0.0)
    o_ref[...] = (jnp.dot(h, wlast_ref[...], preferred_element_type=jnp.float32)
                  + blast_ref[...])


_RB = 1000


def _dense(parts, x, w_rel, w_root, b):
    grid = (_N // _RB,)
    return pl.pallas_call(
        _dense_body,
        grid=grid,
        in_specs=[
            pl.BlockSpec((_NC, _RB, _D), lambda i: (0, i, 0)),
            pl.BlockSpec((_RB, _D), lambda i: (i, 0)),
            pl.BlockSpec((_D, _D), lambda i: (0, 0)),
            pl.BlockSpec((_D, _D), lambda i: (0, 0)),
            pl.BlockSpec((1, _D), lambda i: (0, 0)),
        ],
        out_specs=pl.BlockSpec((_RB, _D), lambda i: (i, 0)),
        out_shape=jax.ShapeDtypeStruct((_N, _D), jnp.float32),
    )(parts, x, w_rel, w_root, b.reshape(1, _D))


def _head(parts, x, w_rel, w_root, b, w_fc, b_fc, w_last_p, b_last_p):
    grid = (_N // _RB,)
    return pl.pallas_call(
        _head_body,
        grid=grid,
        in_specs=[
            pl.BlockSpec((_NC, _RB, _D), lambda i: (0, i, 0)),
            pl.BlockSpec((_RB, _D), lambda i: (i, 0)),
            pl.BlockSpec((_D, _D), lambda i: (0, 0)),
            pl.BlockSpec((_D, _D), lambda i: (0, 0)),
            pl.BlockSpec((1, _D), lambda i: (0, 0)),
            pl.BlockSpec((_D, _D), lambda i: (0, 0)),
            pl.BlockSpec((1, _D), lambda i: (0, 0)),
            pl.BlockSpec((_D, _D), lambda i: (0, 0)),
            pl.BlockSpec((1, _D), lambda i: (0, 0)),
        ],
        out_specs=pl.BlockSpec((_RB, _D), lambda i: (i, 0)),
        out_shape=jax.ShapeDtypeStruct((_N, _D), jnp.float32),
    )(parts, x, w_rel, w_root, b.reshape(1, _D), w_fc, b_fc.reshape(1, _D),
      w_last_p, b_last_p)


def kernel(x, edge_index, edge_weights, W_rel1, b_rel1, W_root1,
           W_rel2, b_rel2, W_root2, W_fc, b_fc, W_last, b_last):
    pad = _EPAD - _E
    spread = (jnp.arange(pad, dtype=jnp.int32) % _N)
    src = jnp.concatenate(
        [edge_index[0].astype(jnp.int32), spread]).reshape(_NW, _C, _K)
    dst = jnp.concatenate(
        [edge_index[1].astype(jnp.int32), spread]).reshape(_NW, _C, _K)
    w = jnp.concatenate(
        [edge_weights, jnp.zeros((pad,), jnp.float32)]
    ).reshape(_NW, _C, _K)

    w_last_p = jnp.zeros((_D, _D), jnp.float32).at[:, :_HOR].set(W_last)
    b_last_p = jnp.zeros((1, _D), jnp.float32).at[0, :_HOR].set(b_last)

    p1 = _spmm(x, src, dst, w)
    h1 = _dense(p1, x, W_rel1, W_root1, b_rel1)
    p2 = _spmm(h1, src, dst, w)
    h2 = _dense(p2, h1, W_rel2, W_root2, b_rel2)
    p3 = _spmm(h2, src, dst, w)
    out = _head(p3, h2, W_rel2, W_root2, b_rel2, W_fc, b_fc,
                w_last_p, b_last_p)
    return out[:, :_HOR]
